# baseline (speedup 1.0000x reference)
import functools

import jax
import jax.numpy as jnp
from jax import lax
from jax.experimental import pallas as pl
from jax.experimental.pallas import tpu as pltpu
from jax.experimental.pallas import tpu_sc as plsc

_W0 = 1000.0
_H0 = 1000.0

_NC = 2
_NS = 16
_NW = _NC * _NS
_CHUNK = 128


@functools.lru_cache(maxsize=None)
def _build(B, D):
    e_per_w = B * D // _NW
    assert (B * D) % _NW == 0 and e_per_w % _CHUNK == 0
    n_chunks = e_per_w // _CHUNK

    mesh = plsc.VectorSubcoreMesh(core_axis_name="c", subcore_axis_name="s")

    @functools.partial(
        pl.kernel,
        mesh=mesh,
        out_type=[
            jax.ShapeDtypeStruct((B * D,), jnp.float32),
            jax.ShapeDtypeStruct((B * D,), jnp.float32),
            jax.ShapeDtypeStruct((16,), jnp.float32),
        ],
        scratch_types=[
            pltpu.VMEM((e_per_w,), jnp.int32),
            pltpu.VMEM((e_per_w,), jnp.float32),
            pltpu.VMEM((e_per_w,), jnp.float32),
            pltpu.VMEM((16,), jnp.float32),
            pltpu.SemaphoreType.DMA,
            pltpu.SemaphoreType.DMA,
        ],
    )
    def gather_kernel(phi_hbm, t_hbm, f_hbm, idx_hbm,
                      phi_out, t_out, f_out,
                      exp_v, phi_rows, t_rows, f_v, sem_p, sem_t):
        wid = lax.axis_index("s") * _NC + lax.axis_index("c")
        base = wid * e_per_w

        pltpu.sync_copy(idx_hbm.at[pl.ds(base, e_per_w)], exp_v)

        copies = []
        for j in range(n_chunks):
            sl = pl.ds(j * _CHUNK, _CHUNK)
            copies.append(
                pltpu.async_copy(phi_hbm.at[exp_v.at[sl]], phi_rows.at[sl], sem_p))
            copies.append(
                pltpu.async_copy(t_hbm.at[exp_v.at[sl]], t_rows.at[sl], sem_t))
        for c in copies:
            c.wait()

        pltpu.sync_copy(phi_rows, phi_out.at[pl.ds(base, e_per_w)])
        pltpu.sync_copy(t_rows, t_out.at[pl.ds(base, e_per_w)])

        @pl.when(wid == 0)
        def _():
            pltpu.sync_copy(f_hbm, f_v)
            fv = f_v[...]
            scale = jnp.where(lax.iota(jnp.int32, 16) == 0,
                              jnp.float32(_W0), jnp.float32(_H0))
            f_v[...] = fv * fv * scale
            pltpu.sync_copy(f_v, f_out)

    return gather_kernel


def kernel(phi, t, f, indices):
    B = indices.shape[0]
    D = phi.shape[1]
    idx_exp = (indices.astype(jnp.int32)[:, None] * D
               + jnp.arange(D, dtype=jnp.int32)[None, :]).reshape(-1)
    f16 = jnp.zeros((16,), jnp.float32).at[:2].set(f.astype(jnp.float32))
    phi_sel, t_sel, fxy = _build(B, D)(
        phi.reshape(-1), t.reshape(-1), f16, idx_exp)
    return (phi_sel.reshape(B, D), t_sel.reshape(B, D), fxy[0], fxy[1])


# speedup vs baseline: 63.1971x; 63.1971x over previous
import functools

import jax
import jax.numpy as jnp
from jax import lax
from jax.experimental import pallas as pl
from jax.experimental.pallas import tpu as pltpu
from jax.experimental.pallas import tpu_sc as plsc

_W0 = 1000.0
_H0 = 1000.0

_NC = 2
_NS = 16
_NW = _NC * _NS
_CHUNK = 128


@functools.lru_cache(maxsize=None)
def _build(B, D):
    b_per_w = B // _NW
    assert B % _NW == 0 and b_per_w % _CHUNK == 0
    n_chunks = b_per_w // _CHUNK

    mesh = plsc.VectorSubcoreMesh(core_axis_name="c", subcore_axis_name="s")

    @functools.partial(
        pl.kernel,
        mesh=mesh,
        out_type=[
            jax.ShapeDtypeStruct((D, B), jnp.float32),
            jax.ShapeDtypeStruct((D, B), jnp.float32),
            jax.ShapeDtypeStruct((16,), jnp.float32),
        ],
        scratch_types=[
            pltpu.VMEM((n_chunks, _CHUNK), jnp.int32),
            pltpu.VMEM((D, b_per_w), jnp.float32),
            pltpu.VMEM((D, b_per_w), jnp.float32),
            pltpu.VMEM((16,), jnp.float32),
            pltpu.SemaphoreType.DMA,
            pltpu.SemaphoreType.DMA,
        ],
        compiler_params=pltpu.CompilerParams(use_tc_tiling_on_sc=False),
    )
    def gather_kernel(phi_hbm, t_hbm, f_hbm, idx_hbm,
                      phi_out, t_out, f_out,
                      idx_v, phi_rows, t_rows, f_v, sem_p, sem_t):
        wid = lax.axis_index("s") * _NC + lax.axis_index("c")
        base = wid * b_per_w

        pltpu.sync_copy(idx_hbm.at[pl.ds(wid * n_chunks, n_chunks)], idx_v)

        copies = []
        for c in range(D):
            for j in range(n_chunks):
                sl = pl.ds(j * _CHUNK, _CHUNK)
                copies.append(pltpu.async_copy(
                    phi_hbm.at[c].at[idx_v.at[j]], phi_rows.at[c].at[sl], sem_p))
                copies.append(pltpu.async_copy(
                    t_hbm.at[c].at[idx_v.at[j]], t_rows.at[c].at[sl], sem_t))
        for cp in copies:
            cp.wait()

        pltpu.sync_copy(phi_rows, phi_out.at[:, pl.ds(base, b_per_w)])
        pltpu.sync_copy(t_rows, t_out.at[:, pl.ds(base, b_per_w)])

        @pl.when(wid == 0)
        def _():
            pltpu.sync_copy(f_hbm, f_v)
            fv = f_v[...]
            scale = jnp.where(lax.iota(jnp.int32, 16) == 0,
                              jnp.float32(_W0), jnp.float32(_H0))
            f_v[...] = fv * fv * scale
            pltpu.sync_copy(f_v, f_out)

    return gather_kernel


def kernel(phi, t, f, indices):
    B = indices.shape[0]
    D = phi.shape[1]
    idx2 = indices.astype(jnp.int32).reshape(B // _CHUNK, _CHUNK)
    f16 = jnp.zeros((16,), jnp.float32).at[:2].set(f.astype(jnp.float32))
    phi_sel, t_sel, fxy = _build(B, D)(phi.T, t.T, f16, idx2)
    return (phi_sel.T, t_sel.T, fxy[0], fxy[1])


# speedup vs baseline: 65.5803x; 1.0377x over previous
import functools

import jax
import jax.numpy as jnp
from jax import lax
from jax.experimental import pallas as pl
from jax.experimental.pallas import tpu as pltpu
from jax.experimental.pallas import tpu_sc as plsc

_W0 = 1000.0
_H0 = 1000.0

_NC = 2
_NS = 16
_NW = _NC * _NS
_CHUNK = 128


@functools.lru_cache(maxsize=None)
def _build(B, D, with_f):
    b_per_w = B // _NW
    assert B % _NW == 0 and b_per_w % _CHUNK == 0
    n_chunks = b_per_w // _CHUNK

    mesh = plsc.VectorSubcoreMesh(core_axis_name="c", subcore_axis_name="s")

    out_type = [jax.ShapeDtypeStruct((D, B), jnp.float32)]
    if with_f:
        out_type.append(jax.ShapeDtypeStruct((16,), jnp.float32))
    scratch = [
        pltpu.VMEM((n_chunks, _CHUNK), jnp.int32),
        pltpu.VMEM((D, b_per_w), jnp.float32),
        pltpu.SemaphoreType.DMA,
    ]
    if with_f:
        scratch.append(pltpu.VMEM((16,), jnp.float32))

    def body(*refs):
        if with_f:
            (tab_hbm, f_hbm, idx_hbm, tab_out, f_out,
             idx_v, rows_v, sem, f_v) = refs
        else:
            tab_hbm, idx_hbm, tab_out, idx_v, rows_v, sem = refs
        wid = lax.axis_index("s") * _NC + lax.axis_index("c")
        base = wid * b_per_w

        pltpu.sync_copy(idx_hbm.at[pl.ds(wid * n_chunks, n_chunks)], idx_v)

        copies = []
        for c in range(D):
            for j in range(n_chunks):
                sl = pl.ds(j * _CHUNK, _CHUNK)
                copies.append(pltpu.async_copy(
                    tab_hbm.at[c].at[idx_v.at[j]], rows_v.at[c].at[sl], sem))
        for cp in copies:
            cp.wait()

        pltpu.sync_copy(rows_v, tab_out.at[:, pl.ds(base, b_per_w)])

        if with_f:
            @pl.when(wid == 0)
            def _():
                pltpu.sync_copy(f_hbm, f_v)
                fv = f_v[...]
                scale = jnp.where(lax.iota(jnp.int32, 16) == 0,
                                  jnp.float32(_W0), jnp.float32(_H0))
                f_v[...] = fv * fv * scale
                pltpu.sync_copy(f_v, f_out)

    return pl.kernel(
        body,
        mesh=mesh,
        out_type=out_type,
        scratch_types=scratch,
        compiler_params=pltpu.CompilerParams(use_tc_tiling_on_sc=False),
    )


def kernel(phi, t, f, indices):
    B = indices.shape[0]
    D = phi.shape[1]
    idx2 = indices.astype(jnp.int32).reshape(B // _CHUNK, _CHUNK)
    f16 = jnp.zeros((16,), jnp.float32).at[:2].set(f.astype(jnp.float32))
    phi_sel, fxy = _build(B, D, True)(phi.T, f16, idx2)
    t_sel, = _build(B, D, False)(t.T, idx2)
    return (phi_sel.T, t_sel.T, fxy[0], fxy[1])
